# Initial kernel scaffold; baseline (speedup 1.0000x reference)
import functools

import jax
import jax.numpy as jnp
from jax import lax
from jax.experimental import pallas as pl
from jax.experimental.pallas import tpu as pltpu
from jax.experimental.pallas import tpu_sc as plsc

NUM_ROWS = 16384 * 50
DIM = 32
NC, NS = 2, 16
NW = NC * NS
BPW = NUM_ROWS // NW
IW = 128
NROWS_IDX = BPW // IW
K = 8
CHUNK = K * IW
NCHUNKS = BPW // CHUNK


def _body(idx_hbm, table_hbm, out_hbm, idx_v, rows_v, sem):
    c = lax.axis_index("c")
    s = lax.axis_index("s")
    wid = s * NC + c
    pltpu.sync_copy(idx_hbm.at[wid], idx_v)

    def chunk(i, _):
        copies = []
        for j in range(K):
            copies.append(pltpu.async_copy(
                table_hbm.at[idx_v.at[i * K + j]],
                rows_v.at[pl.ds(j * IW, IW)],
                sem,
            ))
        for cp in copies:
            cp.wait()
        pltpu.sync_copy(rows_v, out_hbm.at[pl.ds(wid * BPW + i * CHUNK, CHUNK)])
        return 0

    lax.fori_loop(0, NCHUNKS, chunk, 0)


@functools.partial(
    pl.kernel,
    mesh=plsc.VectorSubcoreMesh(core_axis_name="c", subcore_axis_name="s"),
    out_type=jax.ShapeDtypeStruct((NUM_ROWS, DIM), jnp.float32),
    scratch_types=[
        pltpu.VMEM((NROWS_IDX, IW), jnp.int32),
        pltpu.VMEM((CHUNK, DIM), jnp.float32),
        pltpu.SemaphoreType.DMA,
    ],
)
def _gather_kernel(idx_hbm, table_hbm, out_hbm, idx_v, rows_v, sem):
    _body(idx_hbm, table_hbm, out_hbm, idx_v, rows_v, sem)


def kernel(token_ids, weights):
    idx = token_ids.reshape(NW, NROWS_IDX, IW).astype(jnp.int32)
    out = _gather_kernel(idx, weights)
    return out.reshape(token_ids.shape + (DIM,))


# speedup vs baseline: 1.1033x; 1.1033x over previous
import functools

import jax
import jax.numpy as jnp
from jax import lax
from jax.experimental import pallas as pl
from jax.experimental.pallas import tpu as pltpu
from jax.experimental.pallas import tpu_sc as plsc

NUM_ROWS = 16384 * 50
DIM = 32
NC, NS = 2, 16
NW = NC * NS
BPW = NUM_ROWS // NW
IW = 128
NROWS_IDX = BPW // IW
K = 8
CHUNK = K * IW
NCHUNKS = BPW // CHUNK


def _body(idx_hbm, table_hbm, out_hbm, idx_v, rows_v, sem):
    c = lax.axis_index("c")
    s = lax.axis_index("s")
    wid = s * NC + c
    pltpu.sync_copy(idx_hbm.at[wid], idx_v)

    def chunk(i, _):
        copies = []
        for j in range(K):
            copies.append(pltpu.async_copy(
                table_hbm.at[idx_v.at[i * K + j]],
                rows_v.at[pl.ds(j * IW, IW)],
                sem,
            ))
        for cp in copies:
            cp.wait()
        pltpu.sync_copy(rows_v, out_hbm.at[pl.ds(wid * BPW + i * CHUNK, CHUNK)])
        return 0

    lax.fori_loop(0, NCHUNKS, chunk, 0)


@functools.partial(
    pl.kernel,
    mesh=plsc.VectorSubcoreMesh(core_axis_name="c", subcore_axis_name="s"),
    compiler_params=pltpu.CompilerParams(use_tc_tiling_on_sc=False),
    out_type=jax.ShapeDtypeStruct((NUM_ROWS, DIM), jnp.float32),
    scratch_types=[
        pltpu.VMEM((NROWS_IDX, IW), jnp.int32),
        pltpu.VMEM((CHUNK, DIM), jnp.float32),
        pltpu.SemaphoreType.DMA,
    ],
)
def _gather_kernel(idx_hbm, table_hbm, out_hbm, idx_v, rows_v, sem):
    _body(idx_hbm, table_hbm, out_hbm, idx_v, rows_v, sem)


def kernel(token_ids, weights):
    idx = token_ids.reshape(NW, NROWS_IDX, IW).astype(jnp.int32)
    out = _gather_kernel(idx, weights)
    return out.reshape(token_ids.shape + (DIM,))


# speedup vs baseline: 1.1130x; 1.0088x over previous
import functools

import jax
import jax.numpy as jnp
from jax import lax
from jax.experimental import pallas as pl
from jax.experimental.pallas import tpu as pltpu
from jax.experimental.pallas import tpu_sc as plsc

NUM_ROWS = 16384 * 50
DIM = 32
NC, NS = 2, 16
NW = NC * NS
BPW = NUM_ROWS // NW
IW = 128
NROWS_IDX = BPW // IW
K = 10
CHUNK = K * IW
NCHUNKS = BPW // CHUNK
NGROUPS = NCHUNKS // 2


def _body(idx_hbm, table_hbm, out_hbm, idx_v, rows0, rows1, sem0, sem1):
    c = lax.axis_index("c")
    s = lax.axis_index("s")
    wid = s * NC + c
    base = wid * BPW
    rows = (rows0, rows1)
    sems = (sem0, sem1)

    pltpu.sync_copy(idx_hbm.at[wid], idx_v)

    def fire(i, b):
        for j in range(K):
            pltpu.async_copy(
                table_hbm.at[idx_v.at[i * K + j]],
                rows[b].at[pl.ds(j * IW, IW)],
                sems[b],
            )

    def drain_and_write(i, b):
        for j in range(K):
            pltpu.make_async_copy(
                table_hbm.at[idx_v.at[j]],
                rows[b].at[pl.ds(j * IW, IW)],
                sems[b],
            ).wait()
        pltpu.sync_copy(rows[b], out_hbm.at[pl.ds(base + i * CHUNK, CHUNK)])

    fire(0, 0)

    def group(g, _):
        i0 = g * 2
        fire(i0 + 1, 1)
        drain_and_write(i0, 0)
        fire(i0 + 2, 0)
        drain_and_write(i0 + 1, 1)
        return 0

    lax.fori_loop(0, NGROUPS - 1, group, 0)

    i0 = (NGROUPS - 1) * 2
    fire(i0 + 1, 1)
    drain_and_write(i0, 0)
    drain_and_write(i0 + 1, 1)


@functools.partial(
    pl.kernel,
    mesh=plsc.VectorSubcoreMesh(core_axis_name="c", subcore_axis_name="s"),
    compiler_params=pltpu.CompilerParams(use_tc_tiling_on_sc=False),
    out_type=jax.ShapeDtypeStruct((NUM_ROWS, DIM), jnp.float32),
    scratch_types=[
        pltpu.VMEM((NROWS_IDX, IW), jnp.int32),
        pltpu.VMEM((CHUNK, DIM), jnp.float32),
        pltpu.VMEM((CHUNK, DIM), jnp.float32),
        pltpu.SemaphoreType.DMA,
        pltpu.SemaphoreType.DMA,
    ],
)
def _gather_kernel(idx_hbm, table_hbm, out_hbm, idx_v, rows0, rows1, sem0, sem1):
    _body(idx_hbm, table_hbm, out_hbm, idx_v, rows0, rows1, sem0, sem1)


def kernel(token_ids, weights):
    idx = token_ids.reshape(NW, NROWS_IDX, IW).astype(jnp.int32)
    out = _gather_kernel(idx, weights)
    return out.reshape(token_ids.shape + (DIM,))


# speedup vs baseline: 1.6439x; 1.4770x over previous
import functools

import jax
import jax.numpy as jnp
from jax import lax
from jax.experimental import pallas as pl
from jax.experimental.pallas import tpu as pltpu
from jax.experimental.pallas import tpu_sc as plsc

B_ROWS = 16384
SEQ = 50
DIM = 32
NC, NS = 2, 16
NW = NC * NS
RPW = B_ROWS // NW
NBLK = RPW // 128
IW = 128


def _splat(v):
    return jnp.full((16,), v, jnp.int32)


def _body(idx_hbm, table_hbm, out_hbm, idx_v, rows0, rows1, tb0, tb1,
          gs0, gs1, ws0, ws1):
    c = lax.axis_index("c")
    s = lax.axis_index("s")
    wid = s * NC + c
    pltpu.sync_copy(idx_hbm.at[wid], idx_v)

    rows = (rows0, rows1)
    tbs = (tb0, tb1)
    gsems = (gs0, gs1)
    wsems = (ws0, ws1)
    iota = lax.iota(jnp.int32, 16)

    def fire(b, u):
        for r1l in range(NBLK):
            pltpu.async_copy(
                table_hbm.at[idx_v.at[b * NBLK + r1l]],
                rows[u].at[pl.ds(r1l * IW, IW)],
                gsems[u],
            )

    def wait_gathers(u):
        for r1l in range(NBLK):
            pltpu.make_async_copy(
                table_hbm.at[idx_v.at[r1l]],
                rows[u].at[pl.ds(r1l * IW, IW)],
                gsems[u],
            ).wait()

    def transpose(u):
        def tr(j, _):
            ri = j * 16 + iota
            for r1l in range(NBLK):
                rv = ri + r1l * IW
                for c1 in range(4):
                    for c0 in range(8):
                        c = c1 * 8 + c0
                        v = plsc.load_gather(rows[u], [rv, _splat(c)])
                        tbs[u][c1 * 32 + r1l * 8 + c0,
                               pl.ds(j * 16, 16)] = v
            return 0

        lax.fori_loop(0, 8, tr, 0)

    def fire_writes(b, u):
        for c1 in range(4):
            pltpu.async_copy(
                tbs[u].at[pl.ds(c1 * 32, 32)],
                out_hbm.at[b, c1, pl.ds(32 * wid, 32)],
                wsems[u],
            )

    def wait_writes(u):
        for c1 in range(4):
            pltpu.make_async_copy(
                tbs[u].at[pl.ds(c1 * 32, 32)],
                out_hbm.at[0, c1, pl.ds(32 * wid, 32)],
                wsems[u],
            ).wait()

    fire(0, 0)

    def group(g, _):
        b0 = g * 2

        @pl.when(b0 + 1 < SEQ)
        def _():
            fire(b0 + 1, 1)

        wait_gathers(0)

        @pl.when(g > 0)
        def _():
            wait_writes(0)

        transpose(0)
        fire_writes(b0, 0)

        @pl.when(b0 + 2 < SEQ)
        def _():
            fire(b0 + 2, 0)

        wait_gathers(1)

        @pl.when(g > 0)
        def _():
            wait_writes(1)

        transpose(1)
        fire_writes(b0 + 1, 1)
        return 0

    lax.fori_loop(0, SEQ // 2, group, 0)
    wait_writes(0)
    wait_writes(1)


@functools.partial(
    pl.kernel,
    mesh=plsc.VectorSubcoreMesh(core_axis_name="c", subcore_axis_name="s"),
    compiler_params=pltpu.CompilerParams(
        use_tc_tiling_on_sc=False, needs_layout_passes=False
    ),
    out_type=jax.ShapeDtypeStruct((SEQ, 4, B_ROWS // 16, 128), jnp.float32),
    scratch_types=[
        pltpu.VMEM((SEQ * NBLK, IW), jnp.int32),
        pltpu.VMEM((NBLK * IW, DIM), jnp.float32),
        pltpu.VMEM((NBLK * IW, DIM), jnp.float32),
        pltpu.VMEM((128, 128), jnp.float32),
        pltpu.VMEM((128, 128), jnp.float32),
        pltpu.SemaphoreType.DMA,
        pltpu.SemaphoreType.DMA,
        pltpu.SemaphoreType.DMA,
        pltpu.SemaphoreType.DMA,
    ],
)
def _gather_kernel(idx_hbm, table_hbm, out_hbm, idx_v, rows0, rows1,
                   tb0, tb1, gs0, gs1, ws0, ws1):
    _body(idx_hbm, table_hbm, out_hbm, idx_v, rows0, rows1, tb0, tb1,
          gs0, gs1, ws0, ws1)


def kernel(token_ids, weights):
    idx = (token_ids.astype(jnp.int32)
           .reshape(NW, NBLK, 128, SEQ)
           .transpose(0, 3, 1, 2)
           .reshape(NW, SEQ * NBLK, IW))
    out4 = _gather_kernel(idx, weights)
    out5 = out4.reshape(SEQ, 4, B_ROWS // 128, 8, 128)
    return out5.transpose(2, 4, 0, 1, 3).reshape(B_ROWS, SEQ, DIM)


# speedup vs baseline: 3.0232x; 1.8391x over previous
import functools

import jax
import jax.numpy as jnp
from jax import lax
from jax.experimental import pallas as pl
from jax.experimental.pallas import tpu as pltpu
from jax.experimental.pallas import tpu_sc as plsc

B_ROWS = 16384
SEQ = 50
DIM = 32
NC, NS = 2, 16
NW = NC * NS
RPW = B_ROWS // NW
NBLK = RPW // 128
IW = 128
TBP = 129


def _body(idx_hbm, table_hbm, out_hbm, idx_v, rows0, rows1, tb0, tb1,
          gs0, gs1, ws0, ws1):
    c = lax.axis_index("c")
    s = lax.axis_index("s")
    wid = s * NC + c
    pltpu.sync_copy(idx_hbm.at[wid], idx_v)

    rows = (rows0, rows1)
    tbs = (tb0, tb1)
    gsems = (gs0, gs1)
    wsems = (ws0, ws1)
    iota = lax.iota(jnp.int32, 16)
    rowperm = (iota // 8) * 32 + (iota % 8)

    def fire(b, u):
        for r1l in range(NBLK):
            pltpu.async_copy(
                table_hbm.at[idx_v.at[b * NBLK + r1l]],
                rows[u].at[pl.ds(r1l * IW, IW)],
                gsems[u],
            )

    def wait_gathers(u):
        for r1l in range(NBLK):
            pltpu.make_async_copy(
                table_hbm.at[idx_v.at[r1l]],
                rows[u].at[pl.ds(r1l * IW, IW)],
                gsems[u],
            ).wait()

    def transpose(u):
        for r1l in range(NBLK):
            rv0 = rowperm + r1l * 8
            rv1 = rowperm + (r1l * 8 + 64)

            def tr(jj, _):
                base = r1l * IW + jj * 16
                vs = [(rows[u][base + k, pl.ds(0, 16)],
                       rows[u][base + k, pl.ds(16, 16)])
                      for k in range(16)]
                for k, (v0, v1) in enumerate(vs):
                    lane = jnp.zeros((16,), jnp.int32) + (jj * 16 + k)
                    plsc.store_scatter(tbs[u], [rv0, lane], v0)
                    plsc.store_scatter(tbs[u], [rv1, lane], v1)
                return 0

            lax.fori_loop(0, 8, tr, 0)

    def fire_writes(b, u):
        for c1 in range(4):
            pltpu.async_copy(
                tbs[u].at[pl.ds(c1 * 32, 32), pl.ds(0, 128)],
                out_hbm.at[b, c1, pl.ds(32 * wid, 32)],
                wsems[u],
            )

    def wait_writes(u):
        for c1 in range(4):
            pltpu.make_async_copy(
                tbs[u].at[pl.ds(c1 * 32, 32), pl.ds(0, 128)],
                out_hbm.at[0, c1, pl.ds(32 * wid, 32)],
                wsems[u],
            ).wait()

    fire(0, 0)

    def group(g, _):
        b0 = g * 2

        @pl.when(b0 + 1 < SEQ)
        def _():
            fire(b0 + 1, 1)

        wait_gathers(0)

        @pl.when(g > 0)
        def _():
            wait_writes(0)

        transpose(0)
        fire_writes(b0, 0)

        @pl.when(b0 + 2 < SEQ)
        def _():
            fire(b0 + 2, 0)

        wait_gathers(1)

        @pl.when(g > 0)
        def _():
            wait_writes(1)

        transpose(1)
        fire_writes(b0 + 1, 1)
        return 0

    lax.fori_loop(0, SEQ // 2, group, 0)
    wait_writes(0)
    wait_writes(1)


@functools.partial(
    pl.kernel,
    mesh=plsc.VectorSubcoreMesh(core_axis_name="c", subcore_axis_name="s"),
    compiler_params=pltpu.CompilerParams(
        use_tc_tiling_on_sc=False, needs_layout_passes=False
    ),
    out_type=jax.ShapeDtypeStruct((SEQ, 4, B_ROWS // 16, 128), jnp.float32),
    scratch_types=[
        pltpu.VMEM((SEQ * NBLK, IW), jnp.int32),
        pltpu.VMEM((NBLK * IW, DIM), jnp.float32),
        pltpu.VMEM((NBLK * IW, DIM), jnp.float32),
        pltpu.VMEM((128, TBP), jnp.float32),
        pltpu.VMEM((128, TBP), jnp.float32),
        pltpu.SemaphoreType.DMA,
        pltpu.SemaphoreType.DMA,
        pltpu.SemaphoreType.DMA,
        pltpu.SemaphoreType.DMA,
    ],
)
def _gather_kernel(idx_hbm, table_hbm, out_hbm, idx_v, rows0, rows1,
                   tb0, tb1, gs0, gs1, ws0, ws1):
    _body(idx_hbm, table_hbm, out_hbm, idx_v, rows0, rows1, tb0, tb1,
          gs0, gs1, ws0, ws1)


def kernel(token_ids, weights):
    idx = (token_ids.astype(jnp.int32)
           .reshape(NW, NBLK, 128, SEQ)
           .transpose(0, 3, 1, 2)
           .reshape(NW, SEQ * NBLK, IW))
    out4 = _gather_kernel(idx, weights)
    out5 = out4.reshape(SEQ, 4, B_ROWS // 128, 8, 128)
    return out5.transpose(2, 4, 0, 1, 3).reshape(B_ROWS, SEQ, DIM)
